# Initial kernel scaffold; baseline (speedup 1.0000x reference)
import jax
import jax.numpy as jnp
from jax.experimental import pallas as pl


def kernel(features, features_k, gt_labels, gt_labels_k, memory_features):
    raise NotImplementedError("write your pallas kernel here")


# speedup vs baseline: 3.5966x; 3.5966x over previous
import functools

import jax
import jax.numpy as jnp
from jax.experimental import pallas as pl
from jax.experimental.pallas import tpu as pltpu

NUM_LABELED = 15080
OUT_CHANNELS = 2048
TEMP = 0.05
BATCH = 64

TILE = 512
NTILES = pl.cdiv(NUM_LABELED, TILE)


def _ce_body(feat_ref, tgt_ref, mem_ref, out_ref, m_ref, s_ref, p_ref):
    t = pl.program_id(0)

    @pl.when(t == 0)
    def _init():
        m_ref[...] = jnp.full((BATCH, 1), -jnp.inf, jnp.float32)
        s_ref[...] = jnp.zeros((BATCH, 1), jnp.float32)
        p_ref[...] = jnp.zeros((BATCH, 1), jnp.float32)

    feat = feat_ref[...]
    mem = mem_ref[...]
    logits = jax.lax.dot_general(
        feat, mem, (((1,), (1,)), ((), ())),
        preferred_element_type=jnp.float32,
    ) / TEMP

    col = t * TILE + jax.lax.broadcasted_iota(jnp.int32, (BATCH, TILE), 1)
    valid = col < NUM_LABELED
    logits = jnp.where(valid, logits, -jnp.inf)

    m_old = m_ref[...]
    m_new = jnp.maximum(m_old, jnp.max(logits, axis=1, keepdims=True))
    e = jnp.exp(logits - m_new)
    s_ref[...] = s_ref[...] * jnp.exp(m_old - m_new) + jnp.sum(
        e, axis=1, keepdims=True)
    m_ref[...] = m_new

    hit = col == tgt_ref[...]
    p_ref[...] += jnp.sum(jnp.where(hit, logits, 0.0), axis=1, keepdims=True)

    @pl.when(t == NTILES - 1)
    def _fini():
        lse = m_ref[...] + jnp.log(s_ref[...])
        out_ref[0, 0] = jnp.mean(lse - p_ref[...])


@functools.partial(jax.jit, static_argnames=("interpret",))
def _ce_loss(feat, targets, memory_features, interpret=False):
    out = pl.pallas_call(
        _ce_body,
        grid=(NTILES,),
        in_specs=[
            pl.BlockSpec((BATCH, OUT_CHANNELS), lambda t: (0, 0)),
            pl.BlockSpec((BATCH, 1), lambda t: (0, 0)),
            pl.BlockSpec((TILE, OUT_CHANNELS), lambda t: (t, 0)),
        ],
        out_specs=pl.BlockSpec(memory_space=pltpu.SMEM),
        out_shape=jax.ShapeDtypeStruct((1, 1), jnp.float32),
        scratch_shapes=[
            pltpu.VMEM((BATCH, 1), jnp.float32),
            pltpu.VMEM((BATCH, 1), jnp.float32),
            pltpu.VMEM((BATCH, 1), jnp.float32),
        ],
        interpret=interpret,
    )(feat, targets, memory_features)
    return out[0, 0]


def kernel(features, features_k, gt_labels, gt_labels_k, memory_features):
    pids = gt_labels[0, :, -1]
    mask = pids > -1
    feat = jnp.where(mask[:, None], features, 0.0)
    targets = jnp.where(mask, pids, 0).astype(jnp.int32)[:, None]
    return _ce_loss(feat, targets, memory_features)


# speedup vs baseline: 4.4992x; 1.2509x over previous
import functools

import jax
import jax.numpy as jnp
from jax.experimental import pallas as pl
from jax.experimental.pallas import tpu as pltpu

NUM_LABELED = 15080
OUT_CHANNELS = 2048
TEMP = 0.05
BATCH = 64

TILE = 1160
NTILES = NUM_LABELED // TILE


def _ce_body(feat_ref, tgt_ref, mem_ref, out_ref, m_ref, s_ref, p_ref):
    t = pl.program_id(0)

    @pl.when(t == 0)
    def _init():
        m_ref[...] = jnp.full((BATCH, 1), -jnp.inf, jnp.float32)
        s_ref[...] = jnp.zeros((BATCH, 1), jnp.float32)
        p_ref[...] = jnp.zeros((BATCH, 1), jnp.float32)

    feat = feat_ref[...]
    mem = mem_ref[...]
    logits = jax.lax.dot_general(
        feat, mem, (((1,), (1,)), ((), ())),
        preferred_element_type=jnp.float32,
    )

    col = t * TILE + jax.lax.broadcasted_iota(jnp.int32, (BATCH, TILE), 1)

    m_old = m_ref[...]
    m_new = jnp.maximum(m_old, jnp.max(logits, axis=1, keepdims=True))
    e = jnp.exp(logits - m_new)
    s_ref[...] = s_ref[...] * jnp.exp(m_old - m_new) + jnp.sum(
        e, axis=1, keepdims=True)
    m_ref[...] = m_new

    hit = col == tgt_ref[...]
    p_ref[...] += jnp.sum(jnp.where(hit, logits, 0.0), axis=1, keepdims=True)

    @pl.when(t == NTILES - 1)
    def _fini():
        lse = m_ref[...] + jnp.log(s_ref[...])
        out_ref[0, 0] = jnp.mean(lse - p_ref[...])


@functools.partial(jax.jit, static_argnames=("interpret",))
def _ce_loss(feat, targets, memory_features, interpret=False):
    out = pl.pallas_call(
        _ce_body,
        grid=(NTILES,),
        in_specs=[
            pl.BlockSpec((BATCH, OUT_CHANNELS), lambda t: (0, 0)),
            pl.BlockSpec((BATCH, 1), lambda t: (0, 0)),
            pl.BlockSpec((TILE, OUT_CHANNELS), lambda t: (t, 0)),
        ],
        out_specs=pl.BlockSpec(memory_space=pltpu.SMEM),
        out_shape=jax.ShapeDtypeStruct((1, 1), jnp.float32),
        scratch_shapes=[
            pltpu.VMEM((BATCH, 1), jnp.float32),
            pltpu.VMEM((BATCH, 1), jnp.float32),
            pltpu.VMEM((BATCH, 1), jnp.float32),
        ],
        interpret=interpret,
    )(feat, targets, memory_features)
    return out[0, 0]


def kernel(features, features_k, gt_labels, gt_labels_k, memory_features):
    pids = gt_labels[0, :, -1]
    mask = pids > -1
    feat = jnp.where(mask[:, None], features / TEMP, 0.0)
    targets = jnp.where(mask, pids, 0).astype(jnp.int32)[:, None]
    return _ce_loss(feat, targets, memory_features)
